# baseline (speedup 1.0000x reference)
import functools

import numpy as np

import jax
import jax.numpy as jnp
from jax import lax
from jax.experimental import pallas as pl
from jax.experimental.pallas import tpu as pltpu
from jax.experimental.pallas import tpu_sc as plsc

_HI = jax.lax.Precision.HIGHEST
_NC = 2
_NS = 16


def _knn_body(nsrc, x1_ref, x2_ref, idx_ref, w_ref):
    b = pl.program_id(0)
    a = x1_ref[0]
    c = x2_ref[0]
    na = jnp.sum(a * a, axis=1, keepdims=True)
    nc = jnp.sum(c * c, axis=0, keepdims=True)
    ab = a.astype(jnp.bfloat16)
    cbm2 = (-2.0 * c).astype(jnp.bfloat16)
    crossm2 = jnp.dot(ab, cbm2, preferred_element_type=jnp.float32)
    d2 = (na + nc) + crossm2
    lane = jax.lax.broadcasted_iota(jnp.int32, d2.shape, 1).astype(jnp.float32)
    fsrc = jnp.float32(nsrc)
    dcur = d2
    iks = []
    dks = []
    for _ in range(3):
        m = jnp.min(dcur, axis=1, keepdims=True)
        ik = jnp.min(jnp.where(dcur == m, lane, fsrc), axis=1, keepdims=True)
        dks.append(jnp.sqrt(jnp.maximum(m, 0.0)))
        dcur = jnp.where(lane == ik, jnp.float32(jnp.inf), dcur)
        iks.append(ik)
    ws = [1.0 / (dk + 1e-10) for dk in dks]
    wsum = ws[0] + ws[1] + ws[2]
    idx_ref[0] = (jnp.concatenate(iks, axis=1).astype(jnp.int32)
                  + b * nsrc)
    w_ref[0] = jnp.concatenate([w / wsum for w in ws], axis=1)


def _sc_gather_interp(table, idx3, w16, R, D2, G, CHUNKS):
    rows_per_chunk = 3 * G
    FC = D2 // 32
    H = D2 // 2
    mesh = plsc.VectorSubcoreMesh(core_axis_name="c", subcore_axis_name="s")

    @functools.partial(
        pl.kernel,
        mesh=mesh,
        out_type=jax.ShapeDtypeStruct((R, D2), jnp.float32),
        scratch_types=[
            pltpu.VMEM((CHUNKS, rows_per_chunk), jnp.int32),
            pltpu.VMEM((4, rows_per_chunk, 16), jnp.float32),
            pltpu.VMEM((4, rows_per_chunk, D2 // 2), jnp.int32),
            pltpu.VMEM((2, G, D2), jnp.float32),
            pltpu.SemaphoreType.DMA,
            pltpu.SemaphoreType.DMA,
            pltpu.SemaphoreType.DMA,
            pltpu.SemaphoreType.DMA,
            pltpu.SemaphoreType.DMA,
            pltpu.SemaphoreType.DMA,
        ],
    )
    def k(table_hbm, idx_hbm, w16_hbm, out_hbm, idx_v, w16_v, rows_v, out_v,
          sg0, sg1, sg2, sg3, so0, so1):
        wid = lax.axis_index("s") * _NC + lax.axis_index("c")
        base_chunk = wid * CHUNKS
        sgs = (sg0, sg1, sg2, sg3)
        sos = (so0, so1)
        pltpu.sync_copy(idx_hbm.at[wid], idx_v)

        def start(c, buf):
            pltpu.async_copy(table_hbm.at[idx_v.at[c]], rows_v.at[buf],
                             sgs[buf])
            pltpu.async_copy(w16_hbm.at[base_chunk + c], w16_v.at[buf],
                             sgs[buf])

        def wait_gather(c, buf):
            pltpu.make_async_copy(table_hbm.at[idx_v.at[c]], rows_v.at[buf],
                                  sgs[buf]).wait()
            pltpu.make_async_copy(w16_hbm.at[base_chunk + c], w16_v.at[buf],
                                  sgs[buf]).wait()

        def compute(buf, obuf):
            def g_body(g, carry2):
                w0 = w16_v[buf, 3 * g + 0]
                w1 = w16_v[buf, 3 * g + 1]
                w2 = w16_v[buf, 3 * g + 2]
                for fc in range(FC):
                    sl = pl.ds(fc * 16, 16)
                    u0 = rows_v[buf, 3 * g + 0, sl]
                    u1 = rows_v[buf, 3 * g + 1, sl]
                    u2 = rows_v[buf, 3 * g + 2, sl]
                    bc = lambda v: lax.bitcast_convert_type(v, jnp.float32)
                    mhi = jnp.int32(-65536)
                    lo0, hi0 = bc(u0 << 16), bc(u0 & mhi)
                    lo1, hi1 = bc(u1 << 16), bc(u1 & mhi)
                    lo2, hi2 = bc(u2 << 16), bc(u2 & mhi)
                    out_v[obuf, g, pl.ds(fc * 16, 16)] = (
                        lo0 * w0 + lo1 * w1 + lo2 * w2)
                    out_v[obuf, g, pl.ds(H + fc * 16, 16)] = (
                        hi0 * w0 + hi1 * w1 + hi2 * w2)
                return carry2

            lax.fori_loop(0, G, g_body, 0)

        def out_start(c, obuf):
            pltpu.async_copy(
                out_v.at[obuf],
                out_hbm.at[pl.ds(wid * (CHUNKS * G) + c * G, G)], sos[obuf])

        def out_wait(obuf):
            pltpu.make_async_copy(out_v.at[obuf], out_hbm.at[pl.ds(0, G)],
                                  sos[obuf]).wait()

        start(0, 0)
        start(1, 1)
        start(2, 2)

        def loop_body(i, carry):
            base = 4 * i
            for j in range(4):
                c = base + j
                obuf = j % 2

                @pl.when(c + 3 < CHUNKS)
                def _():
                    start(c + 3, (j + 3) % 4)

                wait_gather(c, j)

                @pl.when(c >= 2)
                def _():
                    out_wait(obuf)

                compute(j, obuf)
                out_start(c, obuf)
            return carry

        lax.fori_loop(0, CHUNKS // 4, loop_body, 0)
        out_wait(0)
        out_wait(1)

    return k(table, idx3, w16)


def _mlp1_body(x1_ref, x2_ref, w1a_ref, w1b_ref, b1_ref, y_ref, s_ref, q_ref):
    i = pl.program_id(0)
    y = (lax.dot_general(x1_ref[0], w1a_ref[...], (((0,), (0,)), ((), ())),
                         preferred_element_type=jnp.float32)
         + jnp.dot(x2_ref[...], w1b_ref[...], preferred_element_type=jnp.float32)
         + b1_ref[...])
    y_ref[...] = y

    @pl.when(i == 0)
    def _():
        s_ref[...] = jnp.zeros_like(s_ref)
        q_ref[...] = jnp.zeros_like(q_ref)

    s_ref[...] += jnp.sum(y, axis=0, keepdims=True)
    q_ref[...] += jnp.sum(y * y, axis=0, keepdims=True)


def _mlp2_body(count, part_steps, y1a_ref, y1b_ref, y1c_ref, y1d_ref,
               s1_ref, q1_ref, g1_ref, be1_ref, w2_ref, b2_ref,
               y2_ref, s_ref, q_ref):
    i = pl.program_id(0)
    y1 = jnp.where(
        i < part_steps, y1a_ref[...],
        jnp.where(i < 2 * part_steps, y1b_ref[...],
                  jnp.where(i < 3 * part_steps, y1c_ref[...], y1d_ref[...])))
    inv_n = jnp.float32(1.0 / count)
    mean = s1_ref[...] * inv_n
    var = q1_ref[...] * inv_n - mean * mean
    scale = g1_ref[...] / jnp.sqrt(var + 1e-5)
    h = jnp.maximum((y1 - mean) * scale + be1_ref[...], 0.0)
    y2 = jnp.dot(h, w2_ref[...], preferred_element_type=jnp.float32) + b2_ref[...]
    y2_ref[...] = y2

    @pl.when(i == 0)
    def _():
        s_ref[...] = jnp.zeros_like(s_ref)
        q_ref[...] = jnp.zeros_like(q_ref)

    s_ref[...] += jnp.sum(y2, axis=0, keepdims=True)
    q_ref[...] += jnp.sum(y2 * y2, axis=0, keepdims=True)


def _final_body(count, y2_ref, s2_ref, q2_ref, g2_ref, be2_ref, out_ref):
    inv_n = jnp.float32(1.0 / count)
    mean = s2_ref[...] * inv_n
    var = q2_ref[...] * inv_n - mean * mean
    scale = g2_ref[...] / jnp.sqrt(var + 1e-5)
    yn = jnp.maximum((y2_ref[...] - mean) * scale + be2_ref[...], 0.0)
    out_ref[0] = yn.T


def kernel(xyz1, xyz2, points1, points2, W1, b1, g1, be1, W2, b2, g2, be2):
    B, _, N = xyz1.shape
    S = xyz2.shape[2]
    D1 = points1.shape[1]
    D2 = points2.shape[1]
    C1 = W1.shape[0]
    C2 = W2.shape[0]

    NB = min(512, N)
    x1t = jnp.transpose(xyz1, (0, 2, 1))
    p2t = jnp.transpose(points2, (0, 2, 1))

    R = B * N
    NW = _NC * _NS
    G = 16
    M = min(512, R)
    Gm = R // M
    nbb = N // M
    w1aT = jnp.transpose(W1[:, :D1])
    perm = np.concatenate([np.arange(0, D2, 2), np.arange(1, D2, 2)])
    w1bT = jnp.transpose(W1[:, D1:])[perm]
    w2T = jnp.transpose(W2)
    row2 = lambda v: v.reshape(1, -1)

    NPART = 4
    BH = B // NPART
    RH = BH * N
    CHUNKS = RH // (NW * G)
    halves = []
    for h in range(NPART):
        sl = slice(h * BH, (h + 1) * BH)
        idx_h, w_h = pl.pallas_call(
            functools.partial(_knn_body, S),
            grid=(BH, N // NB),
            in_specs=[
                pl.BlockSpec((1, NB, 3), lambda b, i: (b, i, 0)),
                pl.BlockSpec((1, 3, S), lambda b, i: (b, 0, 0)),
            ],
            out_specs=[
                pl.BlockSpec((1, NB, 3), lambda b, i: (b, i, 0)),
                pl.BlockSpec((1, NB, 3), lambda b, i: (b, i, 0)),
            ],
            out_shape=[
                jax.ShapeDtypeStruct((BH, N, 3), jnp.int32),
                jax.ShapeDtypeStruct((BH, N, 3), jnp.float32),
            ],
        )(x1t[sl], xyz2[sl])
        idx3_h = idx_h.reshape(NW, CHUNKS, 3 * G)
        w16_h = jnp.broadcast_to(w_h.reshape(RH * 3, 1), (RH * 3, 16)).reshape(
            NW * CHUNKS, 3 * G, 16)
        table_h = lax.bitcast_convert_type(
            p2t[sl].astype(jnp.bfloat16).reshape(BH * S, D2 // 2, 2),
            jnp.int32)
        interp_h = _sc_gather_interp(table_h, idx3_h, w16_h, RH, D2, G, CHUNKS)
        y1_h, s1_h, q1_h = pl.pallas_call(
            _mlp1_body,
            grid=(RH // M,),
            in_specs=[
                pl.BlockSpec((1, D1, M), lambda i: (i // nbb, 0, i % nbb)),
                pl.BlockSpec((M, D2), lambda i: (i, 0)),
                pl.BlockSpec((D1, C1), lambda i: (0, 0)),
                pl.BlockSpec((D2, C1), lambda i: (0, 0)),
                pl.BlockSpec((1, C1), lambda i: (0, 0)),
            ],
            out_specs=[
                pl.BlockSpec((M, C1), lambda i: (i, 0)),
                pl.BlockSpec((1, C1), lambda i: (0, 0)),
                pl.BlockSpec((1, C1), lambda i: (0, 0)),
            ],
            out_shape=[
                jax.ShapeDtypeStruct((RH, C1), jnp.float32),
                jax.ShapeDtypeStruct((1, C1), jnp.float32),
                jax.ShapeDtypeStruct((1, C1), jnp.float32),
            ],
        )(points1[sl], interp_h, w1aT, w1bT, row2(b1))
        halves.append((y1_h, s1_h, q1_h))

    y1s = [t[0] for t in halves]
    s1 = halves[0][1] + halves[1][1] + halves[2][1] + halves[3][1]
    q1 = halves[0][2] + halves[1][2] + halves[2][2] + halves[3][2]

    Gh = Gm // 4
    y2, s2, q2 = pl.pallas_call(
        functools.partial(_mlp2_body, R, Gh),
        grid=(Gm,),
        in_specs=[
            pl.BlockSpec((M, C1), lambda i: (i % Gh, 0)),
            pl.BlockSpec((M, C1), lambda i: (i % Gh, 0)),
            pl.BlockSpec((M, C1), lambda i: (i % Gh, 0)),
            pl.BlockSpec((M, C1), lambda i: (i % Gh, 0)),
            pl.BlockSpec((1, C1), lambda i: (0, 0)),
            pl.BlockSpec((1, C1), lambda i: (0, 0)),
            pl.BlockSpec((1, C1), lambda i: (0, 0)),
            pl.BlockSpec((1, C1), lambda i: (0, 0)),
            pl.BlockSpec((C1, C2), lambda i: (0, 0)),
            pl.BlockSpec((1, C2), lambda i: (0, 0)),
        ],
        out_specs=[
            pl.BlockSpec((M, C2), lambda i: (i, 0)),
            pl.BlockSpec((1, C2), lambda i: (0, 0)),
            pl.BlockSpec((1, C2), lambda i: (0, 0)),
        ],
        out_shape=[
            jax.ShapeDtypeStruct((R, C2), jnp.float32),
            jax.ShapeDtypeStruct((1, C2), jnp.float32),
            jax.ShapeDtypeStruct((1, C2), jnp.float32),
        ],
    )(y1s[0], y1s[1], y1s[2], y1s[3], s1, q1, row2(g1), row2(be1), w2T,
      row2(b2))

    out = pl.pallas_call(
        functools.partial(_final_body, R),
        grid=(Gm,),
        in_specs=[
            pl.BlockSpec((M, C2), lambda i: (i, 0)),
            pl.BlockSpec((1, C2), lambda i: (0, 0)),
            pl.BlockSpec((1, C2), lambda i: (0, 0)),
            pl.BlockSpec((1, C2), lambda i: (0, 0)),
            pl.BlockSpec((1, C2), lambda i: (0, 0)),
        ],
        out_specs=pl.BlockSpec((1, C2, M), lambda i: (i // nbb, 0, i % nbb)),
        out_shape=jax.ShapeDtypeStruct((B, C2, N), jnp.float32),
    )(y2, s2, q2, row2(g2), row2(be2))

    return out


# speedup vs baseline: 1.0671x; 1.0671x over previous
import functools

import numpy as np

import jax
import jax.numpy as jnp
from jax import lax
from jax.experimental import pallas as pl
from jax.experimental.pallas import tpu as pltpu
from jax.experimental.pallas import tpu_sc as plsc

_HI = jax.lax.Precision.HIGHEST
_NC = 2
_NS = 16


def _knn_body(nsrc, x1_ref, x2_ref, idx_ref, w_ref):
    b = pl.program_id(0)
    a = x1_ref[0]
    c = x2_ref[0]
    na = jnp.sum(a * a, axis=1, keepdims=True)
    nc = jnp.sum(c * c, axis=0, keepdims=True)
    ab = a.astype(jnp.bfloat16)
    cbm2 = (-2.0 * c).astype(jnp.bfloat16)
    crossm2 = jnp.dot(ab, cbm2, preferred_element_type=jnp.float32)
    d2 = (na + nc) + crossm2
    lane = jax.lax.broadcasted_iota(jnp.int32, d2.shape, 1).astype(jnp.float32)
    fsrc = jnp.float32(nsrc)
    dcur = d2
    iks = []
    dks = []
    for _ in range(3):
        m = jnp.min(dcur, axis=1, keepdims=True)
        ik = jnp.min(jnp.where(dcur == m, lane, fsrc), axis=1, keepdims=True)
        dks.append(jnp.sqrt(jnp.maximum(m, 0.0)))
        dcur = jnp.where(lane == ik, jnp.float32(jnp.inf), dcur)
        iks.append(ik)
    ws = [1.0 / (dk + 1e-10) for dk in dks]
    wsum = ws[0] + ws[1] + ws[2]
    idx_ref[0] = (jnp.concatenate(iks, axis=1).astype(jnp.int32)
                  + b * nsrc)
    w_ref[0] = jnp.concatenate([w / wsum for w in ws], axis=1)


def _sc_gather_interp(table, idx3, w16, R, D2, G, CHUNKS):
    rows_per_chunk = 3 * G
    FC = D2 // 32
    H = D2 // 2
    mesh = plsc.VectorSubcoreMesh(core_axis_name="c", subcore_axis_name="s")

    @functools.partial(
        pl.kernel,
        mesh=mesh,
        out_type=jax.ShapeDtypeStruct((R, D2), jnp.float32),
        scratch_types=[
            pltpu.VMEM((CHUNKS, rows_per_chunk), jnp.int32),
            pltpu.VMEM((4, rows_per_chunk, 16), jnp.float32),
            pltpu.VMEM((4, rows_per_chunk, D2 // 2), jnp.int32),
            pltpu.VMEM((2, G, D2), jnp.float32),
            pltpu.SemaphoreType.DMA,
            pltpu.SemaphoreType.DMA,
            pltpu.SemaphoreType.DMA,
            pltpu.SemaphoreType.DMA,
            pltpu.SemaphoreType.DMA,
            pltpu.SemaphoreType.DMA,
        ],
    )
    def k(table_hbm, idx_hbm, w16_hbm, out_hbm, idx_v, w16_v, rows_v, out_v,
          sg0, sg1, sg2, sg3, so0, so1):
        wid = lax.axis_index("s") * _NC + lax.axis_index("c")
        base_chunk = wid * CHUNKS
        sgs = (sg0, sg1, sg2, sg3)
        sos = (so0, so1)
        pltpu.sync_copy(idx_hbm.at[wid], idx_v)

        def start(c, buf):
            pltpu.async_copy(table_hbm.at[idx_v.at[c]], rows_v.at[buf],
                             sgs[buf])
            pltpu.async_copy(w16_hbm.at[base_chunk + c], w16_v.at[buf],
                             sgs[buf])

        def wait_gather(c, buf):
            pltpu.make_async_copy(table_hbm.at[idx_v.at[c]], rows_v.at[buf],
                                  sgs[buf]).wait()
            pltpu.make_async_copy(w16_hbm.at[base_chunk + c], w16_v.at[buf],
                                  sgs[buf]).wait()

        def compute(buf, obuf):
            def g_body(g, carry2):
                w0 = w16_v[buf, 3 * g + 0]
                w1 = w16_v[buf, 3 * g + 1]
                w2 = w16_v[buf, 3 * g + 2]
                for fc in range(FC):
                    sl = pl.ds(fc * 16, 16)
                    u0 = rows_v[buf, 3 * g + 0, sl]
                    u1 = rows_v[buf, 3 * g + 1, sl]
                    u2 = rows_v[buf, 3 * g + 2, sl]
                    bc = lambda v: lax.bitcast_convert_type(v, jnp.float32)
                    mhi = jnp.int32(-65536)
                    lo0, hi0 = bc(u0 << 16), bc(u0 & mhi)
                    lo1, hi1 = bc(u1 << 16), bc(u1 & mhi)
                    lo2, hi2 = bc(u2 << 16), bc(u2 & mhi)
                    out_v[obuf, g, pl.ds(fc * 16, 16)] = (
                        lo0 * w0 + lo1 * w1 + lo2 * w2)
                    out_v[obuf, g, pl.ds(H + fc * 16, 16)] = (
                        hi0 * w0 + hi1 * w1 + hi2 * w2)
                return carry2

            lax.fori_loop(0, G, g_body, 0)

        def out_start(c, obuf):
            pltpu.async_copy(
                out_v.at[obuf],
                out_hbm.at[pl.ds(wid * (CHUNKS * G) + c * G, G)], sos[obuf])

        def out_wait(obuf):
            pltpu.make_async_copy(out_v.at[obuf], out_hbm.at[pl.ds(0, G)],
                                  sos[obuf]).wait()

        start(0, 0)
        start(1, 1)
        start(2, 2)

        def loop_body(i, carry):
            base = 4 * i
            for j in range(4):
                c = base + j
                obuf = j % 2

                @pl.when(c + 3 < CHUNKS)
                def _():
                    start(c + 3, (j + 3) % 4)

                wait_gather(c, j)

                @pl.when(c >= 2)
                def _():
                    out_wait(obuf)

                compute(j, obuf)
                out_start(c, obuf)
            return carry

        lax.fori_loop(0, CHUNKS // 4, loop_body, 0)
        out_wait(0)
        out_wait(1)

    return k(table, idx3, w16)


def _mlp1_body(x1_ref, x2_ref, w1a_ref, w1b_ref, b1_ref, y_ref, s_ref, q_ref):
    i = pl.program_id(0)
    y = (lax.dot_general(x1_ref[0], w1a_ref[...], (((0,), (0,)), ((), ())),
                         preferred_element_type=jnp.float32)
         + jnp.dot(x2_ref[...], w1b_ref[...], preferred_element_type=jnp.float32)
         + b1_ref[...])
    y_ref[...] = y.astype(jnp.bfloat16)

    @pl.when(i == 0)
    def _():
        s_ref[...] = jnp.zeros_like(s_ref)
        q_ref[...] = jnp.zeros_like(q_ref)

    s_ref[...] += jnp.sum(y, axis=0, keepdims=True)
    q_ref[...] += jnp.sum(y * y, axis=0, keepdims=True)


def _mlp2_body(count, half_steps, y1a_ref, y1b_ref, s1_ref, q1_ref, g1_ref,
               be1_ref, w2_ref, b2_ref, y2_ref, s_ref, q_ref):
    i = pl.program_id(0)
    y1 = jnp.where(i < half_steps, y1a_ref[...],
                   y1b_ref[...]).astype(jnp.float32)
    inv_n = jnp.float32(1.0 / count)
    mean = s1_ref[...] * inv_n
    var = q1_ref[...] * inv_n - mean * mean
    scale = g1_ref[...] / jnp.sqrt(var + 1e-5)
    h = jnp.maximum((y1 - mean) * scale + be1_ref[...], 0.0)
    y2 = jnp.dot(h, w2_ref[...], preferred_element_type=jnp.float32) + b2_ref[...]
    y2_ref[...] = y2.astype(jnp.bfloat16)

    @pl.when(i == 0)
    def _():
        s_ref[...] = jnp.zeros_like(s_ref)
        q_ref[...] = jnp.zeros_like(q_ref)

    s_ref[...] += jnp.sum(y2, axis=0, keepdims=True)
    q_ref[...] += jnp.sum(y2 * y2, axis=0, keepdims=True)


def _final_body(count, y2_ref, s2_ref, q2_ref, g2_ref, be2_ref, out_ref):
    inv_n = jnp.float32(1.0 / count)
    mean = s2_ref[...] * inv_n
    var = q2_ref[...] * inv_n - mean * mean
    scale = g2_ref[...] / jnp.sqrt(var + 1e-5)
    y2f = y2_ref[...].astype(jnp.float32)
    yn = jnp.maximum((y2f - mean) * scale + be2_ref[...], 0.0)
    out_ref[0] = yn.T


def kernel(xyz1, xyz2, points1, points2, W1, b1, g1, be1, W2, b2, g2, be2):
    B, _, N = xyz1.shape
    S = xyz2.shape[2]
    D1 = points1.shape[1]
    D2 = points2.shape[1]
    C1 = W1.shape[0]
    C2 = W2.shape[0]

    NB = min(512, N)
    x1t = jnp.transpose(xyz1, (0, 2, 1))
    p2t = jnp.transpose(points2, (0, 2, 1))

    R = B * N
    NW = _NC * _NS
    G = 16
    M = min(512, R)
    Gm = R // M
    nbb = N // M
    w1aT = jnp.transpose(W1[:, :D1])
    perm = np.concatenate([np.arange(0, D2, 2), np.arange(1, D2, 2)])
    w1bT = jnp.transpose(W1[:, D1:])[perm]
    w2T = jnp.transpose(W2)
    row2 = lambda v: v.reshape(1, -1)

    BH = B // 2
    RH = BH * N
    CHUNKS = RH // (NW * G)
    halves = []
    for h in range(2):
        sl = slice(h * BH, (h + 1) * BH)
        idx_h, w_h = pl.pallas_call(
            functools.partial(_knn_body, S),
            grid=(BH, N // NB),
            in_specs=[
                pl.BlockSpec((1, NB, 3), lambda b, i: (b, i, 0)),
                pl.BlockSpec((1, 3, S), lambda b, i: (b, 0, 0)),
            ],
            out_specs=[
                pl.BlockSpec((1, NB, 3), lambda b, i: (b, i, 0)),
                pl.BlockSpec((1, NB, 3), lambda b, i: (b, i, 0)),
            ],
            out_shape=[
                jax.ShapeDtypeStruct((BH, N, 3), jnp.int32),
                jax.ShapeDtypeStruct((BH, N, 3), jnp.float32),
            ],
        )(x1t[sl], xyz2[sl])
        idx3_h = idx_h.reshape(NW, CHUNKS, 3 * G)
        w16_h = jnp.broadcast_to(w_h.reshape(RH * 3, 1), (RH * 3, 16)).reshape(
            NW * CHUNKS, 3 * G, 16)
        table_h = lax.bitcast_convert_type(
            p2t[sl].astype(jnp.bfloat16).reshape(BH * S, D2 // 2, 2),
            jnp.int32)
        interp_h = _sc_gather_interp(table_h, idx3_h, w16_h, RH, D2, G, CHUNKS)
        y1_h, s1_h, q1_h = pl.pallas_call(
            _mlp1_body,
            grid=(Gm // 2,),
            in_specs=[
                pl.BlockSpec((1, D1, M), lambda i: (i // nbb, 0, i % nbb)),
                pl.BlockSpec((M, D2), lambda i: (i, 0)),
                pl.BlockSpec((D1, C1), lambda i: (0, 0)),
                pl.BlockSpec((D2, C1), lambda i: (0, 0)),
                pl.BlockSpec((1, C1), lambda i: (0, 0)),
            ],
            out_specs=[
                pl.BlockSpec((M, C1), lambda i: (i, 0)),
                pl.BlockSpec((1, C1), lambda i: (0, 0)),
                pl.BlockSpec((1, C1), lambda i: (0, 0)),
            ],
            out_shape=[
                jax.ShapeDtypeStruct((RH, C1), jnp.bfloat16),
                jax.ShapeDtypeStruct((1, C1), jnp.float32),
                jax.ShapeDtypeStruct((1, C1), jnp.float32),
            ],
        )(points1[sl], interp_h, w1aT, w1bT, row2(b1))
        halves.append((y1_h, s1_h, q1_h))

    (y1a, s1a, q1a), (y1b, s1b, q1b) = halves
    s1 = s1a + s1b
    q1 = q1a + q1b

    Gh = Gm // 2
    y2, s2, q2 = pl.pallas_call(
        functools.partial(_mlp2_body, R, Gh),
        grid=(Gm,),
        in_specs=[
            pl.BlockSpec((M, C1), lambda i: (i % Gh, 0)),
            pl.BlockSpec((M, C1), lambda i: (i % Gh, 0)),
            pl.BlockSpec((1, C1), lambda i: (0, 0)),
            pl.BlockSpec((1, C1), lambda i: (0, 0)),
            pl.BlockSpec((1, C1), lambda i: (0, 0)),
            pl.BlockSpec((1, C1), lambda i: (0, 0)),
            pl.BlockSpec((C1, C2), lambda i: (0, 0)),
            pl.BlockSpec((1, C2), lambda i: (0, 0)),
        ],
        out_specs=[
            pl.BlockSpec((M, C2), lambda i: (i, 0)),
            pl.BlockSpec((1, C2), lambda i: (0, 0)),
            pl.BlockSpec((1, C2), lambda i: (0, 0)),
        ],
        out_shape=[
            jax.ShapeDtypeStruct((R, C2), jnp.bfloat16),
            jax.ShapeDtypeStruct((1, C2), jnp.float32),
            jax.ShapeDtypeStruct((1, C2), jnp.float32),
        ],
    )(y1a, y1b, s1, q1, row2(g1), row2(be1), w2T, row2(b2))

    out = pl.pallas_call(
        functools.partial(_final_body, R),
        grid=(Gm,),
        in_specs=[
            pl.BlockSpec((M, C2), lambda i: (i, 0)),
            pl.BlockSpec((1, C2), lambda i: (0, 0)),
            pl.BlockSpec((1, C2), lambda i: (0, 0)),
            pl.BlockSpec((1, C2), lambda i: (0, 0)),
            pl.BlockSpec((1, C2), lambda i: (0, 0)),
        ],
        out_specs=pl.BlockSpec((1, C2, M), lambda i: (i // nbb, 0, i % nbb)),
        out_shape=jax.ShapeDtypeStruct((B, C2, N), jnp.float32),
    )(y2, s2, q2, row2(g2), row2(be2))

    return out


# speedup vs baseline: 1.0761x; 1.0085x over previous
import functools

import numpy as np

import jax
import jax.numpy as jnp
from jax import lax
from jax.experimental import pallas as pl
from jax.experimental.pallas import tpu as pltpu
from jax.experimental.pallas import tpu_sc as plsc

_HI = jax.lax.Precision.HIGHEST
_NC = 2
_NS = 16


def _knn_body(nsrc, x1_ref, x2_ref, idx_ref, w_ref):
    b = pl.program_id(0)
    a = x1_ref[0]
    c = x2_ref[0]
    na = jnp.sum(a * a, axis=1, keepdims=True)
    nc = jnp.sum(c * c, axis=0, keepdims=True)
    ab = a.astype(jnp.bfloat16)
    cbm2 = (-2.0 * c).astype(jnp.bfloat16)
    crossm2 = jnp.dot(ab, cbm2, preferred_element_type=jnp.float32)
    d2 = (na + nc) + crossm2
    lane = jax.lax.broadcasted_iota(jnp.int32, d2.shape, 1).astype(jnp.float32)
    fsrc = jnp.float32(nsrc)
    dcur = d2
    iks = []
    dks = []
    for _ in range(3):
        m = jnp.min(dcur, axis=1, keepdims=True)
        ik = jnp.min(jnp.where(dcur == m, lane, fsrc), axis=1, keepdims=True)
        dks.append(jnp.sqrt(jnp.maximum(m, 0.0)))
        dcur = jnp.where(lane == ik, jnp.float32(jnp.inf), dcur)
        iks.append(ik)
    ws = [1.0 / (dk + 1e-10) for dk in dks]
    wsum = ws[0] + ws[1] + ws[2]
    idx_ref[0] = (jnp.concatenate(iks, axis=1).astype(jnp.int32)
                  + b * nsrc)
    w_ref[0] = jnp.concatenate([w / wsum for w in ws], axis=1)


def _sc_gather_interp(table, idx3, w16, R, D2, G, CHUNKS):
    rows_per_chunk = 3 * G
    FC = D2 // 32
    H = D2 // 2
    mesh = plsc.VectorSubcoreMesh(core_axis_name="c", subcore_axis_name="s")

    @functools.partial(
        pl.kernel,
        mesh=mesh,
        out_type=jax.ShapeDtypeStruct((R, D2), jnp.float32),
        scratch_types=[
            pltpu.VMEM((CHUNKS, rows_per_chunk), jnp.int32),
            pltpu.VMEM((4, rows_per_chunk, 16), jnp.float32),
            pltpu.VMEM((4, rows_per_chunk, D2 // 2), jnp.int32),
            pltpu.VMEM((2, G, D2), jnp.float32),
            pltpu.SemaphoreType.DMA,
            pltpu.SemaphoreType.DMA,
            pltpu.SemaphoreType.DMA,
            pltpu.SemaphoreType.DMA,
            pltpu.SemaphoreType.DMA,
            pltpu.SemaphoreType.DMA,
        ],
    )
    def k(table_hbm, idx_hbm, w16_hbm, out_hbm, idx_v, w16_v, rows_v, out_v,
          sg0, sg1, sg2, sg3, so0, so1):
        wid = lax.axis_index("s") * _NC + lax.axis_index("c")
        base_chunk = wid * CHUNKS
        sgs = (sg0, sg1, sg2, sg3)
        sos = (so0, so1)
        pltpu.sync_copy(idx_hbm.at[wid], idx_v)

        def start(c, buf):
            pltpu.async_copy(table_hbm.at[idx_v.at[c]], rows_v.at[buf],
                             sgs[buf])
            pltpu.async_copy(w16_hbm.at[base_chunk + c], w16_v.at[buf],
                             sgs[buf])

        def wait_gather(c, buf):
            pltpu.make_async_copy(table_hbm.at[idx_v.at[c]], rows_v.at[buf],
                                  sgs[buf]).wait()
            pltpu.make_async_copy(w16_hbm.at[base_chunk + c], w16_v.at[buf],
                                  sgs[buf]).wait()

        def compute(buf, obuf):
            def g_body(g, carry2):
                w0 = w16_v[buf, 3 * g + 0]
                w1 = w16_v[buf, 3 * g + 1]
                w2 = w16_v[buf, 3 * g + 2]
                for fc in range(FC):
                    sl = pl.ds(fc * 16, 16)
                    u0 = rows_v[buf, 3 * g + 0, sl]
                    u1 = rows_v[buf, 3 * g + 1, sl]
                    u2 = rows_v[buf, 3 * g + 2, sl]
                    bc = lambda v: lax.bitcast_convert_type(v, jnp.float32)
                    mhi = jnp.int32(-65536)
                    lo0, hi0 = bc(u0 << 16), bc(u0 & mhi)
                    lo1, hi1 = bc(u1 << 16), bc(u1 & mhi)
                    lo2, hi2 = bc(u2 << 16), bc(u2 & mhi)
                    out_v[obuf, g, pl.ds(fc * 16, 16)] = (
                        lo0 * w0 + lo1 * w1 + lo2 * w2)
                    out_v[obuf, g, pl.ds(H + fc * 16, 16)] = (
                        hi0 * w0 + hi1 * w1 + hi2 * w2)
                return carry2

            lax.fori_loop(0, G, g_body, 0)

        def out_start(c, obuf):
            pltpu.async_copy(
                out_v.at[obuf],
                out_hbm.at[pl.ds(wid * (CHUNKS * G) + c * G, G)], sos[obuf])

        def out_wait(obuf):
            pltpu.make_async_copy(out_v.at[obuf], out_hbm.at[pl.ds(0, G)],
                                  sos[obuf]).wait()

        start(0, 0)
        start(1, 1)
        start(2, 2)

        def loop_body(i, carry):
            base = 4 * i
            for j in range(4):
                c = base + j
                obuf = j % 2

                @pl.when(c + 3 < CHUNKS)
                def _():
                    start(c + 3, (j + 3) % 4)

                wait_gather(c, j)

                @pl.when(c >= 2)
                def _():
                    out_wait(obuf)

                compute(j, obuf)
                out_start(c, obuf)
            return carry

        lax.fori_loop(0, CHUNKS // 4, loop_body, 0)
        out_wait(0)
        out_wait(1)

    return k(table, idx3, w16)


def _mlp1_body(x1_ref, x2_ref, w1a_ref, w1b_ref, b1_ref, y_ref, s_ref, q_ref):
    i = pl.program_id(0)
    y = (lax.dot_general(x1_ref[0], w1a_ref[...], (((0,), (0,)), ((), ())),
                         preferred_element_type=jnp.float32)
         + jnp.dot(x2_ref[...], w1b_ref[...], preferred_element_type=jnp.float32)
         + b1_ref[...])
    y_ref[...] = y.astype(jnp.bfloat16)

    @pl.when(i == 0)
    def _():
        s_ref[...] = jnp.zeros_like(s_ref)
        q_ref[...] = jnp.zeros_like(q_ref)

    s_ref[...] += jnp.sum(y, axis=0, keepdims=True)
    q_ref[...] += jnp.sum(y * y, axis=0, keepdims=True)


def _mlp2_body(count, half_steps, y1a_ref, y1b_ref, s1_ref, q1_ref, g1_ref,
               be1_ref, w2_ref, b2_ref, y2_ref, s_ref, q_ref):
    i = pl.program_id(0)
    y1 = jnp.where(i < half_steps, y1a_ref[...],
                   y1b_ref[...]).astype(jnp.float32)
    inv_n = jnp.float32(1.0 / count)
    mean = s1_ref[...] * inv_n
    var = q1_ref[...] * inv_n - mean * mean
    scale = g1_ref[...] / jnp.sqrt(var + 1e-5)
    h = jnp.maximum((y1 - mean) * scale + be1_ref[...], 0.0)
    y2 = jnp.dot(h, w2_ref[...], preferred_element_type=jnp.float32) + b2_ref[...]
    y2_ref[...] = y2.astype(jnp.bfloat16)

    @pl.when(i == 0)
    def _():
        s_ref[...] = jnp.zeros_like(s_ref)
        q_ref[...] = jnp.zeros_like(q_ref)

    s_ref[...] += jnp.sum(y2, axis=0, keepdims=True)
    q_ref[...] += jnp.sum(y2 * y2, axis=0, keepdims=True)


def _final_body(count, y2_ref, s2_ref, q2_ref, g2_ref, be2_ref, out_ref):
    inv_n = jnp.float32(1.0 / count)
    mean = s2_ref[...] * inv_n
    var = q2_ref[...] * inv_n - mean * mean
    scale = g2_ref[...] / jnp.sqrt(var + 1e-5)
    y2f = y2_ref[...].astype(jnp.float32)
    yn = jnp.maximum((y2f - mean) * scale + be2_ref[...], 0.0)
    out_ref[0] = yn.T


def kernel(xyz1, xyz2, points1, points2, W1, b1, g1, be1, W2, b2, g2, be2):
    B, _, N = xyz1.shape
    S = xyz2.shape[2]
    D1 = points1.shape[1]
    D2 = points2.shape[1]
    C1 = W1.shape[0]
    C2 = W2.shape[0]

    NB = min(1024, N)
    x1t = jnp.transpose(xyz1, (0, 2, 1))
    p2t = jnp.transpose(points2, (0, 2, 1))

    R = B * N
    NW = _NC * _NS
    G = 16
    M = min(512, R)
    Gm = R // M
    nbb = N // M
    w1aT = jnp.transpose(W1[:, :D1])
    perm = np.concatenate([np.arange(0, D2, 2), np.arange(1, D2, 2)])
    w1bT = jnp.transpose(W1[:, D1:])[perm]
    w2T = jnp.transpose(W2)
    row2 = lambda v: v.reshape(1, -1)

    BH = B // 2
    RH = BH * N
    CHUNKS = RH // (NW * G)
    halves = []
    for h in range(2):
        sl = slice(h * BH, (h + 1) * BH)
        idx_h, w_h = pl.pallas_call(
            functools.partial(_knn_body, S),
            grid=(BH, N // NB),
            in_specs=[
                pl.BlockSpec((1, NB, 3), lambda b, i: (b, i, 0)),
                pl.BlockSpec((1, 3, S), lambda b, i: (b, 0, 0)),
            ],
            out_specs=[
                pl.BlockSpec((1, NB, 3), lambda b, i: (b, i, 0)),
                pl.BlockSpec((1, NB, 3), lambda b, i: (b, i, 0)),
            ],
            out_shape=[
                jax.ShapeDtypeStruct((BH, N, 3), jnp.int32),
                jax.ShapeDtypeStruct((BH, N, 3), jnp.float32),
            ],
        )(x1t[sl], xyz2[sl])
        idx3_h = idx_h.reshape(NW, CHUNKS, 3 * G)
        w16_h = jnp.broadcast_to(w_h.reshape(RH * 3, 1), (RH * 3, 16)).reshape(
            NW * CHUNKS, 3 * G, 16)
        table_h = lax.bitcast_convert_type(
            p2t[sl].astype(jnp.bfloat16).reshape(BH * S, D2 // 2, 2),
            jnp.int32)
        interp_h = _sc_gather_interp(table_h, idx3_h, w16_h, RH, D2, G, CHUNKS)
        y1_h, s1_h, q1_h = pl.pallas_call(
            _mlp1_body,
            grid=(Gm // 2,),
            in_specs=[
                pl.BlockSpec((1, D1, M), lambda i: (i // nbb, 0, i % nbb)),
                pl.BlockSpec((M, D2), lambda i: (i, 0)),
                pl.BlockSpec((D1, C1), lambda i: (0, 0)),
                pl.BlockSpec((D2, C1), lambda i: (0, 0)),
                pl.BlockSpec((1, C1), lambda i: (0, 0)),
            ],
            out_specs=[
                pl.BlockSpec((M, C1), lambda i: (i, 0)),
                pl.BlockSpec((1, C1), lambda i: (0, 0)),
                pl.BlockSpec((1, C1), lambda i: (0, 0)),
            ],
            out_shape=[
                jax.ShapeDtypeStruct((RH, C1), jnp.bfloat16),
                jax.ShapeDtypeStruct((1, C1), jnp.float32),
                jax.ShapeDtypeStruct((1, C1), jnp.float32),
            ],
        )(points1[sl], interp_h, w1aT, w1bT, row2(b1))
        halves.append((y1_h, s1_h, q1_h))

    (y1a, s1a, q1a), (y1b, s1b, q1b) = halves
    s1 = s1a + s1b
    q1 = q1a + q1b

    Gh = Gm // 2
    y2, s2, q2 = pl.pallas_call(
        functools.partial(_mlp2_body, R, Gh),
        grid=(Gm,),
        in_specs=[
            pl.BlockSpec((M, C1), lambda i: (i % Gh, 0)),
            pl.BlockSpec((M, C1), lambda i: (i % Gh, 0)),
            pl.BlockSpec((1, C1), lambda i: (0, 0)),
            pl.BlockSpec((1, C1), lambda i: (0, 0)),
            pl.BlockSpec((1, C1), lambda i: (0, 0)),
            pl.BlockSpec((1, C1), lambda i: (0, 0)),
            pl.BlockSpec((C1, C2), lambda i: (0, 0)),
            pl.BlockSpec((1, C2), lambda i: (0, 0)),
        ],
        out_specs=[
            pl.BlockSpec((M, C2), lambda i: (i, 0)),
            pl.BlockSpec((1, C2), lambda i: (0, 0)),
            pl.BlockSpec((1, C2), lambda i: (0, 0)),
        ],
        out_shape=[
            jax.ShapeDtypeStruct((R, C2), jnp.bfloat16),
            jax.ShapeDtypeStruct((1, C2), jnp.float32),
            jax.ShapeDtypeStruct((1, C2), jnp.float32),
        ],
    )(y1a, y1b, s1, q1, row2(g1), row2(be1), w2T, row2(b2))

    out = pl.pallas_call(
        functools.partial(_final_body, R),
        grid=(Gm,),
        in_specs=[
            pl.BlockSpec((M, C2), lambda i: (i, 0)),
            pl.BlockSpec((1, C2), lambda i: (0, 0)),
            pl.BlockSpec((1, C2), lambda i: (0, 0)),
            pl.BlockSpec((1, C2), lambda i: (0, 0)),
            pl.BlockSpec((1, C2), lambda i: (0, 0)),
        ],
        out_specs=pl.BlockSpec((1, C2, M), lambda i: (i // nbb, 0, i % nbb)),
        out_shape=jax.ShapeDtypeStruct((B, C2, N), jnp.float32),
    )(y2, s2, q2, row2(g2), row2(be2))

    return out


# speedup vs baseline: 1.1709x; 1.0881x over previous
import functools

import numpy as np

import jax
import jax.numpy as jnp
from jax import lax
from jax.experimental import pallas as pl
from jax.experimental.pallas import tpu as pltpu
from jax.experimental.pallas import tpu_sc as plsc

_HI = jax.lax.Precision.HIGHEST
_NC = 2
_NS = 16


def _knn_body(nsrc, x1_ref, x2_ref, idx_ref, w_ref):
    b = pl.program_id(0)
    a = x1_ref[0]
    c = x2_ref[0]
    na = jnp.sum(a * a, axis=1, keepdims=True)
    nc = jnp.sum(c * c, axis=0, keepdims=True)
    ab = a.astype(jnp.bfloat16)
    cbm2 = (-2.0 * c).astype(jnp.bfloat16)
    crossm2 = jnp.dot(ab, cbm2, preferred_element_type=jnp.float32)
    d2 = (na + nc) + crossm2
    lane = jax.lax.broadcasted_iota(jnp.int32, d2.shape, 1).astype(jnp.float32)
    fsrc = jnp.float32(nsrc)
    dcur = d2
    iks = []
    dks = []
    for _ in range(3):
        m = jnp.min(dcur, axis=1, keepdims=True)
        ik = jnp.min(jnp.where(dcur == m, lane, fsrc), axis=1, keepdims=True)
        dks.append(jnp.sqrt(jnp.maximum(m, 0.0)))
        dcur = jnp.where(lane == ik, jnp.float32(jnp.inf), dcur)
        iks.append(ik)
    ws = [1.0 / (dk + 1e-10) for dk in dks]
    wsum = ws[0] + ws[1] + ws[2]
    idx_ref[0] = (jnp.concatenate(iks, axis=1).astype(jnp.int32)
                  + b * nsrc)
    w_ref[0] = jnp.concatenate([w / wsum for w in ws], axis=1)


def _sc_gather_interp(table, idx3, w16, R, D2, G, CHUNKS):
    rows_per_chunk = 3 * G
    FC = D2 // 32
    H = D2 // 2
    mesh = plsc.VectorSubcoreMesh(core_axis_name="c", subcore_axis_name="s")

    @functools.partial(
        pl.kernel,
        mesh=mesh,
        out_type=jax.ShapeDtypeStruct((R, D2), jnp.float32),
        scratch_types=[
            pltpu.VMEM((CHUNKS, rows_per_chunk), jnp.int32),
            pltpu.VMEM((4, rows_per_chunk, 16), jnp.float32),
            pltpu.VMEM((4, rows_per_chunk, D2 // 2), jnp.int32),
            pltpu.VMEM((2, G, D2), jnp.float32),
            pltpu.SemaphoreType.DMA,
            pltpu.SemaphoreType.DMA,
            pltpu.SemaphoreType.DMA,
            pltpu.SemaphoreType.DMA,
            pltpu.SemaphoreType.DMA,
            pltpu.SemaphoreType.DMA,
        ],
    )
    def k(table_hbm, idx_hbm, w16_hbm, out_hbm, idx_v, w16_v, rows_v, out_v,
          sg0, sg1, sg2, sg3, so0, so1):
        wid = lax.axis_index("s") * _NC + lax.axis_index("c")
        base_chunk = wid * CHUNKS
        sgs = (sg0, sg1, sg2, sg3)
        sos = (so0, so1)
        pltpu.sync_copy(idx_hbm.at[wid], idx_v)

        def start(c, buf):
            pltpu.async_copy(table_hbm.at[idx_v.at[c]], rows_v.at[buf],
                             sgs[buf])
            pltpu.async_copy(w16_hbm.at[base_chunk + c], w16_v.at[buf],
                             sgs[buf])

        def wait_gather(c, buf):
            pltpu.make_async_copy(table_hbm.at[idx_v.at[c]], rows_v.at[buf],
                                  sgs[buf]).wait()
            pltpu.make_async_copy(w16_hbm.at[base_chunk + c], w16_v.at[buf],
                                  sgs[buf]).wait()

        def compute(buf, obuf):
            def g_body(g, carry2):
                w0 = w16_v[buf, 3 * g + 0]
                w1 = w16_v[buf, 3 * g + 1]
                w2 = w16_v[buf, 3 * g + 2]
                for fc in range(FC):
                    sl = pl.ds(fc * 16, 16)
                    u0 = rows_v[buf, 3 * g + 0, sl]
                    u1 = rows_v[buf, 3 * g + 1, sl]
                    u2 = rows_v[buf, 3 * g + 2, sl]
                    bc = lambda v: lax.bitcast_convert_type(v, jnp.float32)
                    mhi = jnp.int32(-65536)
                    lo0, hi0 = bc(u0 << 16), bc(u0 & mhi)
                    lo1, hi1 = bc(u1 << 16), bc(u1 & mhi)
                    lo2, hi2 = bc(u2 << 16), bc(u2 & mhi)
                    out_v[obuf, g, pl.ds(fc * 16, 16)] = (
                        lo0 * w0 + lo1 * w1 + lo2 * w2)
                    out_v[obuf, g, pl.ds(H + fc * 16, 16)] = (
                        hi0 * w0 + hi1 * w1 + hi2 * w2)
                return carry2

            lax.fori_loop(0, G, g_body, 0)

        def out_start(c, obuf):
            pltpu.async_copy(
                out_v.at[obuf],
                out_hbm.at[pl.ds(wid * (CHUNKS * G) + c * G, G)], sos[obuf])

        def out_wait(obuf):
            pltpu.make_async_copy(out_v.at[obuf], out_hbm.at[pl.ds(0, G)],
                                  sos[obuf]).wait()

        start(0, 0)
        start(1, 1)
        start(2, 2)

        def loop_body(i, carry):
            base = 4 * i
            for j in range(4):
                c = base + j
                obuf = j % 2

                @pl.when(c + 3 < CHUNKS)
                def _():
                    start(c + 3, (j + 3) % 4)

                wait_gather(c, j)

                @pl.when(c >= 2)
                def _():
                    out_wait(obuf)

                compute(j, obuf)
                out_start(c, obuf)
            return carry

        lax.fori_loop(0, CHUNKS // 4, loop_body, 0)
        out_wait(0)
        out_wait(1)

    return k(table, idx3, w16)


def _mlp1_body(x1_ref, x2_ref, w1a_ref, w1b_ref, b1_ref, y_ref, s_ref, q_ref):
    i = pl.program_id(0)
    y = (lax.dot_general(x1_ref[0], w1a_ref[...], (((0,), (0,)), ((), ())),
                         preferred_element_type=jnp.float32)
         + jnp.dot(x2_ref[...], w1b_ref[...], preferred_element_type=jnp.float32)
         + b1_ref[...])
    y_ref[...] = y.astype(jnp.bfloat16)

    @pl.when(i == 0)
    def _():
        s_ref[...] = jnp.zeros_like(s_ref)
        q_ref[...] = jnp.zeros_like(q_ref)

    s_ref[...] += jnp.sum(y, axis=0, keepdims=True)
    q_ref[...] += jnp.sum(y * y, axis=0, keepdims=True)


def _mlp2_body(count, half_steps, y1a_ref, y1b_ref, s1_ref, q1_ref, g1_ref,
               be1_ref, w2_ref, b2_ref, y2_ref, s_ref, q_ref):
    i = pl.program_id(0)
    y1 = jnp.where(i < half_steps, y1a_ref[...],
                   y1b_ref[...]).astype(jnp.float32)
    inv_n = jnp.float32(1.0 / count)
    mean = s1_ref[...] * inv_n
    var = q1_ref[...] * inv_n - mean * mean
    scale = g1_ref[...] / jnp.sqrt(var + 1e-5)
    h = jnp.maximum((y1 - mean) * scale + be1_ref[...], 0.0)
    y2 = jnp.dot(h, w2_ref[...], preferred_element_type=jnp.float32) + b2_ref[...]
    y2_ref[...] = y2.astype(jnp.bfloat16)

    @pl.when(i == 0)
    def _():
        s_ref[...] = jnp.zeros_like(s_ref)
        q_ref[...] = jnp.zeros_like(q_ref)

    s_ref[...] += jnp.sum(y2, axis=0, keepdims=True)
    q_ref[...] += jnp.sum(y2 * y2, axis=0, keepdims=True)


def _final_body(count, y2_ref, s2_ref, q2_ref, g2_ref, be2_ref, out_ref):
    inv_n = jnp.float32(1.0 / count)
    mean = s2_ref[...] * inv_n
    var = q2_ref[...] * inv_n - mean * mean
    scale = g2_ref[...] / jnp.sqrt(var + 1e-5)
    y2f = y2_ref[...].astype(jnp.float32)
    yn = jnp.maximum((y2f - mean) * scale + be2_ref[...], 0.0)
    out_ref[0] = yn.T


def kernel(xyz1, xyz2, points1, points2, W1, b1, g1, be1, W2, b2, g2, be2):
    B, _, N = xyz1.shape
    S = xyz2.shape[2]
    D1 = points1.shape[1]
    D2 = points2.shape[1]
    C1 = W1.shape[0]
    C2 = W2.shape[0]

    NB = min(1024, N)
    x1t = jnp.transpose(xyz1, (0, 2, 1))
    p2t = jnp.transpose(points2, (0, 2, 1))

    R = B * N
    NW = _NC * _NS
    G = 16
    M = min(1024, R)
    Gm = R // M
    nbb = N // M
    w1aT = jnp.transpose(W1[:, :D1])
    perm = np.concatenate([np.arange(0, D2, 2), np.arange(1, D2, 2)])
    w1bT = jnp.transpose(W1[:, D1:])[perm]
    w2T = jnp.transpose(W2)
    row2 = lambda v: v.reshape(1, -1)

    BH = B // 2
    RH = BH * N
    CHUNKS = RH // (NW * G)
    halves = []
    for h in range(2):
        sl = slice(h * BH, (h + 1) * BH)
        idx_h, w_h = pl.pallas_call(
            functools.partial(_knn_body, S),
            grid=(BH, N // NB),
            in_specs=[
                pl.BlockSpec((1, NB, 3), lambda b, i: (b, i, 0)),
                pl.BlockSpec((1, 3, S), lambda b, i: (b, 0, 0)),
            ],
            out_specs=[
                pl.BlockSpec((1, NB, 3), lambda b, i: (b, i, 0)),
                pl.BlockSpec((1, NB, 3), lambda b, i: (b, i, 0)),
            ],
            out_shape=[
                jax.ShapeDtypeStruct((BH, N, 3), jnp.int32),
                jax.ShapeDtypeStruct((BH, N, 3), jnp.float32),
            ],
        )(x1t[sl], xyz2[sl])
        idx3_h = idx_h.reshape(NW, CHUNKS, 3 * G)
        w16_h = jnp.broadcast_to(w_h.reshape(RH * 3, 1), (RH * 3, 16)).reshape(
            NW * CHUNKS, 3 * G, 16)
        table_h = lax.bitcast_convert_type(
            p2t[sl].astype(jnp.bfloat16).reshape(BH * S, D2 // 2, 2),
            jnp.int32)
        interp_h = _sc_gather_interp(table_h, idx3_h, w16_h, RH, D2, G, CHUNKS)
        y1_h, s1_h, q1_h = pl.pallas_call(
            _mlp1_body,
            grid=(Gm // 2,),
            in_specs=[
                pl.BlockSpec((1, D1, M), lambda i: (i // nbb, 0, i % nbb)),
                pl.BlockSpec((M, D2), lambda i: (i, 0)),
                pl.BlockSpec((D1, C1), lambda i: (0, 0)),
                pl.BlockSpec((D2, C1), lambda i: (0, 0)),
                pl.BlockSpec((1, C1), lambda i: (0, 0)),
            ],
            out_specs=[
                pl.BlockSpec((M, C1), lambda i: (i, 0)),
                pl.BlockSpec((1, C1), lambda i: (0, 0)),
                pl.BlockSpec((1, C1), lambda i: (0, 0)),
            ],
            out_shape=[
                jax.ShapeDtypeStruct((RH, C1), jnp.bfloat16),
                jax.ShapeDtypeStruct((1, C1), jnp.float32),
                jax.ShapeDtypeStruct((1, C1), jnp.float32),
            ],
        )(points1[sl], interp_h, w1aT, w1bT, row2(b1))
        halves.append((y1_h, s1_h, q1_h))

    (y1a, s1a, q1a), (y1b, s1b, q1b) = halves
    s1 = s1a + s1b
    q1 = q1a + q1b

    Gh = Gm // 2
    y2, s2, q2 = pl.pallas_call(
        functools.partial(_mlp2_body, R, Gh),
        grid=(Gm,),
        in_specs=[
            pl.BlockSpec((M, C1), lambda i: (i % Gh, 0)),
            pl.BlockSpec((M, C1), lambda i: (i % Gh, 0)),
            pl.BlockSpec((1, C1), lambda i: (0, 0)),
            pl.BlockSpec((1, C1), lambda i: (0, 0)),
            pl.BlockSpec((1, C1), lambda i: (0, 0)),
            pl.BlockSpec((1, C1), lambda i: (0, 0)),
            pl.BlockSpec((C1, C2), lambda i: (0, 0)),
            pl.BlockSpec((1, C2), lambda i: (0, 0)),
        ],
        out_specs=[
            pl.BlockSpec((M, C2), lambda i: (i, 0)),
            pl.BlockSpec((1, C2), lambda i: (0, 0)),
            pl.BlockSpec((1, C2), lambda i: (0, 0)),
        ],
        out_shape=[
            jax.ShapeDtypeStruct((R, C2), jnp.bfloat16),
            jax.ShapeDtypeStruct((1, C2), jnp.float32),
            jax.ShapeDtypeStruct((1, C2), jnp.float32),
        ],
    )(y1a, y1b, s1, q1, row2(g1), row2(be1), w2T, row2(b2))

    out = pl.pallas_call(
        functools.partial(_final_body, R),
        grid=(Gm,),
        in_specs=[
            pl.BlockSpec((M, C2), lambda i: (i, 0)),
            pl.BlockSpec((1, C2), lambda i: (0, 0)),
            pl.BlockSpec((1, C2), lambda i: (0, 0)),
            pl.BlockSpec((1, C2), lambda i: (0, 0)),
            pl.BlockSpec((1, C2), lambda i: (0, 0)),
        ],
        out_specs=pl.BlockSpec((1, C2, M), lambda i: (i // nbb, 0, i % nbb)),
        out_shape=jax.ShapeDtypeStruct((B, C2, N), jnp.float32),
    )(y2, s2, q2, row2(g2), row2(be2))

    return out
